# Initial kernel scaffold; baseline (speedup 1.0000x reference)
import jax
import jax.numpy as jnp
from jax.experimental import pallas as pl


def kernel(inputs, codebook):
    raise NotImplementedError("write your pallas kernel here")


# speedup vs baseline: 1.2990x; 1.2990x over previous
import functools

import jax
import jax.numpy as jnp
from jax import lax
from jax.experimental import pallas as pl
from jax.experimental.pallas import tpu as pltpu
from jax.experimental.pallas import tpu_sc as plsc

N_TOKENS = 16384
N_CODES = 8192
DIM = 64
BLOCK_M = 256
GRID_M = N_TOKENS // BLOCK_M
COMMIT = 0.25
EPS = 1e-12


def _argmax_body(x_ref, cb_ref, idx_ref, ncb_ref):
    i = pl.program_id(0)

    @pl.when(i == 0)
    def _():
        cb = cb_ref[...]
        n = jnp.sqrt(jnp.sum(cb * cb, axis=1, keepdims=True))
        ncb_ref[...] = cb / jnp.maximum(n, EPS)

    x = x_ref[...]
    xn = x / jnp.maximum(jnp.sqrt(jnp.sum(x * x, axis=1, keepdims=True)), EPS)
    sims = lax.dot_general(
        xn, ncb_ref[...], (((1,), (1,)), ((), ())),
        preferred_element_type=jnp.float32)
    m = jnp.max(sims, axis=1, keepdims=True)
    col = lax.broadcasted_iota(jnp.int32, sims.shape, 1)
    cand = jnp.where(sims == m, col, jnp.int32(N_CODES))
    idx = jnp.min(cand, axis=1)
    idx_ref[...] = idx.reshape(1, 1, BLOCK_M)


_argmax_call = pl.pallas_call(
    _argmax_body,
    grid=(GRID_M,),
    in_specs=[
        pl.BlockSpec((BLOCK_M, DIM), lambda i: (i, 0)),
        pl.BlockSpec((N_CODES, DIM), lambda i: (0, 0)),
    ],
    out_specs=pl.BlockSpec((1, 1, BLOCK_M), lambda i: (i, 0, 0)),
    out_shape=jax.ShapeDtypeStruct((GRID_M, 1, BLOCK_M), jnp.int32),
    scratch_shapes=[pltpu.VMEM((N_CODES, DIM), jnp.float32)],
)


_NC, _NS = 2, 16
_NW = _NC * _NS
_IDX_ROW = 128
_ROWS_PER_W = N_TOKENS // _NW
_K = _ROWS_PER_W // _IDX_ROW
_DPAD = 128


@functools.cache
def _make_sc_gather():
    @functools.partial(
        pl.kernel,
        mesh=plsc.VectorSubcoreMesh(core_axis_name="c", subcore_axis_name="s"),
        out_type=jax.ShapeDtypeStruct((N_TOKENS, _DPAD), jnp.float32),
        scratch_types=[
            pltpu.VMEM((_K, _IDX_ROW), jnp.int32),
            pltpu.VMEM((_ROWS_PER_W, _DPAD), jnp.float32),
            pltpu.SemaphoreType.DMA,
        ],
    )
    def _sc_gather(table_hbm, idx_hbm, out_hbm, idx_v, rows_v, sem):
        wid = lax.axis_index("s") * _NC + lax.axis_index("c")
        pltpu.sync_copy(idx_hbm.at[pl.ds(wid * _K, _K)], idx_v)
        handles = []
        for k in range(_K):
            handles.append(pltpu.async_copy(
                table_hbm.at[idx_v.at[k]],
                rows_v.at[pl.ds(k * _IDX_ROW, _IDX_ROW)],
                sem))
        for h in handles:
            h.wait()
        pltpu.sync_copy(rows_v,
                        out_hbm.at[pl.ds(wid * _ROWS_PER_W, _ROWS_PER_W)])

    return _sc_gather


def _st_loss_body(q_ref, x_ref, qst_ref, loss_ref):
    q = q_ref[...][:, :DIM]
    x = x_ref[...]
    d = q - x
    qst_ref[...] = x + d
    loss_ref[0, 0] = jnp.sum(d * d) * ((1.0 + COMMIT) / (N_TOKENS * DIM))


_st_loss_call = pl.pallas_call(
    _st_loss_body,
    out_shape=[
        jax.ShapeDtypeStruct((N_TOKENS, DIM), jnp.float32),
        jax.ShapeDtypeStruct((1, 1), jnp.float32),
    ],
    out_specs=[
        pl.BlockSpec(memory_space=pltpu.VMEM),
        pl.BlockSpec(memory_space=pltpu.SMEM),
    ],
)


def kernel(inputs, codebook):
    idx3 = _argmax_call(inputs, codebook)
    idx2 = idx3.reshape(N_TOKENS // _IDX_ROW, _IDX_ROW)
    cb_pad = jnp.pad(codebook, ((0, 0), (0, _DPAD - DIM)))
    quantized_pad = _make_sc_gather()(cb_pad, idx2)
    qst, loss = _st_loss_call(quantized_pad, inputs)
    return (qst, loss.reshape(()))


# speedup vs baseline: 1.8583x; 1.4305x over previous
import functools

import jax
import jax.numpy as jnp
from jax import lax
from jax.experimental import pallas as pl
from jax.experimental.pallas import tpu as pltpu
from jax.experimental.pallas import tpu_sc as plsc

N_TOKENS = 16384
N_CODES = 8192
DIM = 64
BLOCK_M = 256
GRID_M = N_TOKENS // BLOCK_M
COMMIT = 0.25
EPS = 1e-12


def _argmax_body(x_ref, cb_ref, idx_ref, ncb_ref):
    i = pl.program_id(0)

    @pl.when(i == 0)
    def _():
        cb = cb_ref[...]
        n = jnp.sqrt(jnp.sum(cb * cb, axis=1, keepdims=True))
        ncb_ref[...] = cb / jnp.maximum(n, EPS)

    x = x_ref[...]
    xn = x / jnp.maximum(jnp.sqrt(jnp.sum(x * x, axis=1, keepdims=True)), EPS)
    sims = lax.dot_general(
        xn, ncb_ref[...], (((1,), (1,)), ((), ())),
        preferred_element_type=jnp.float32)
    idx = jnp.argmax(sims, axis=1).astype(jnp.int32)
    idx_ref[...] = idx.reshape(1, 1, BLOCK_M)


_argmax_call = pl.pallas_call(
    _argmax_body,
    grid=(GRID_M,),
    in_specs=[
        pl.BlockSpec((BLOCK_M, DIM), lambda i: (i, 0)),
        pl.BlockSpec((N_CODES, DIM), lambda i: (0, 0)),
    ],
    out_specs=pl.BlockSpec((1, 1, BLOCK_M), lambda i: (i, 0, 0)),
    out_shape=jax.ShapeDtypeStruct((GRID_M, 1, BLOCK_M), jnp.int32),
    scratch_shapes=[pltpu.VMEM((N_CODES, DIM), jnp.float32)],
)


_NC, _NS = 2, 16
_NW = _NC * _NS
_IDX_ROW = 128
_ROWS_PER_W = N_TOKENS // _NW
_K = _ROWS_PER_W // _IDX_ROW
_DPAD = 128


@functools.cache
def _make_sc_gather():
    @functools.partial(
        pl.kernel,
        mesh=plsc.VectorSubcoreMesh(core_axis_name="c", subcore_axis_name="s"),
        out_type=jax.ShapeDtypeStruct((N_TOKENS, _DPAD), jnp.float32),
        scratch_types=[
            pltpu.VMEM((_K, _IDX_ROW), jnp.int32),
            pltpu.VMEM((_ROWS_PER_W, _DPAD), jnp.float32),
            pltpu.SemaphoreType.DMA,
        ],
    )
    def _sc_gather(table_hbm, idx_hbm, out_hbm, idx_v, rows_v, sem):
        wid = lax.axis_index("s") * _NC + lax.axis_index("c")
        pltpu.sync_copy(idx_hbm.at[pl.ds(wid * _K, _K)], idx_v)
        handles = []
        for k in range(_K):
            handles.append(pltpu.async_copy(
                table_hbm.at[idx_v.at[k]],
                rows_v.at[pl.ds(k * _IDX_ROW, _IDX_ROW)],
                sem))
        for h in handles:
            h.wait()
        pltpu.sync_copy(rows_v,
                        out_hbm.at[pl.ds(wid * _ROWS_PER_W, _ROWS_PER_W)])

    return _sc_gather


def _st_loss_body(q_ref, x_ref, qst_ref, loss_ref):
    q = q_ref[...][:, :DIM]
    x = x_ref[...]
    d = q - x
    qst_ref[...] = x + d
    loss_ref[0, 0] = jnp.sum(d * d) * ((1.0 + COMMIT) / (N_TOKENS * DIM))


_st_loss_call = pl.pallas_call(
    _st_loss_body,
    out_shape=[
        jax.ShapeDtypeStruct((N_TOKENS, DIM), jnp.float32),
        jax.ShapeDtypeStruct((1, 1), jnp.float32),
    ],
    out_specs=[
        pl.BlockSpec(memory_space=pltpu.VMEM),
        pl.BlockSpec(memory_space=pltpu.SMEM),
    ],
)


def kernel(inputs, codebook):
    idx3 = _argmax_call(inputs, codebook)
    idx2 = idx3.reshape(N_TOKENS // _IDX_ROW, _IDX_ROW)
    cb_pad = jnp.pad(codebook, ((0, 0), (0, _DPAD - DIM)))
    quantized_pad = _make_sc_gather()(cb_pad, idx2)
    qst, loss = _st_loss_call(quantized_pad, inputs)
    return (qst, loss.reshape(()))
